# Initial kernel scaffold; baseline (speedup 1.0000x reference)
import jax
import jax.numpy as jnp
from jax.experimental import pallas as pl


def kernel(xyz, feats, w_qs, w_ks, w_vs, d1_w, d1_b, d2_w, d2_b, g1_w, g1_b, g2_w, g2_b, bn_gamma, bn_beta):
    raise NotImplementedError("write your pallas kernel here")


# speedup vs baseline: 18.7772x; 18.7772x over previous
import jax
import jax.numpy as jnp
from jax.experimental import pallas as pl
from jax.experimental.pallas import tpu as pltpu
from jax.experimental.pallas import tpu_sc as plsc

KNN = 16
TILE = 256
DM = 64
DTBL = 128


def _bits16(x):
    y = x.astype(jnp.bfloat16).astype(jnp.float32)
    return jax.lax.shift_right_logical(
        jax.lax.bitcast_convert_type(y, jnp.int32), 16)


def _unpack_hi(p):
    return jax.lax.bitcast_convert_type(
        jnp.bitwise_and(p, jnp.int32(-65536)), jnp.float32)


def _unpack_lo(p):
    return jax.lax.bitcast_convert_type(
        jax.lax.shift_left(p, 16), jnp.float32)


def _qkv_body(feats_ref, xyz_ref, wq_ref, wk_ref, wv_ref, d1T_ref, d1b_ref,
              qu_ref, tbl_ref):
    f = feats_ref[0]
    qu_ref[0, :, 0:DM] = jnp.dot(f, wq_ref[...], preferred_element_type=jnp.float32)
    k = jnp.dot(f, wk_ref[...], preferred_element_type=jnp.float32)
    v = jnp.dot(f, wv_ref[...], preferred_element_type=jnp.float32)
    x = xyz_ref[0]
    r0 = d1T_ref[0:1, :]
    r1 = d1T_ref[1:2, :]
    r2 = d1T_ref[2:3, :]
    w = x[:, 0:1] * r0 + x[:, 1:2] * r1 + x[:, 2:3] * r2
    qu_ref[0, :, DM:2 * DM] = w + d1b_ref[...]
    whi = w.astype(jnp.bfloat16).astype(jnp.float32)
    wlo = w - whi
    tbl_ref[0, :, 0:DM] = jnp.bitwise_or(
        jax.lax.shift_left(_bits16(k), 16), _bits16(v))
    tbl_ref[0, :, DM:DTBL] = jnp.bitwise_or(
        jax.lax.shift_left(_bits16(whi), 16), _bits16(wlo))


def _knn_body(xyzq_ref, xyzT_ref, knn_ref, dist_ref):
    n = xyzT_ref.shape[2]
    b = pl.program_id(0)
    xq = xyzq_ref[0]
    xk = xyzT_ref[0]
    dx = xq[:, 0:1] - xk[0:1, :]
    dy = xq[:, 1:2] - xk[1:2, :]
    dz = xq[:, 2:3] - xk[2:3, :]
    dist_ref[...] = dx * dx + dy * dy + dz * dz
    jidx = jax.lax.broadcasted_iota(jnp.int32, (TILE, n), 1)
    base = b * n
    for k in range(KNN):
        d = dist_ref[...]
        m = jnp.min(d, axis=1, keepdims=True)
        cand = jnp.where(d == m, jidx, n)
        jmin = jnp.min(cand, axis=1, keepdims=True)
        knn_ref[0, :, k:k + 1] = jmin + base
        dist_ref[...] = jnp.where(jidx == jmin, jnp.inf, d)


def _attn_body(g_ref, qu_ref, feats_ref,
               d2T_ref, d2b_ref,
               g1T_ref, g1b_ref, g2T_ref, g2b_ref, out_ref):
    G = g_ref[0]
    p1 = G[:, 0:DM]
    p2 = G[:, DM:DTBL]
    kg = _unpack_hi(p1).reshape(TILE, KNN, DM)
    vg = _unpack_lo(p1).reshape(TILE, KNN, DM)
    wg = (_unpack_hi(p2) + _unpack_lo(p2)).reshape(TILE, KNN, DM)
    qu = qu_ref[0]
    q3 = qu[:, 0:DM][:, None, :]
    u3 = qu[:, DM:2 * DM][:, None, :]
    pre = u3 - wg
    pe = jax.nn.relu(pre).reshape(TILE * KNN, DM)
    pos = (jnp.dot(pe, d2T_ref[...], preferred_element_type=jnp.float32)
           + d2b_ref[...]).reshape(TILE, KNN, DM)
    h = q3 - kg + pos
    a1 = jax.nn.relu(jnp.dot(h.reshape(TILE * KNN, DM), g1T_ref[...],
                             preferred_element_type=jnp.float32) + g1b_ref[...])
    logits = (jnp.dot(a1, g2T_ref[...], preferred_element_type=jnp.float32)
              + g2b_ref[...]).reshape(TILE, KNN, DM)
    m = jnp.max(logits, axis=1, keepdims=True)
    e = jnp.exp(logits - m)
    s = jnp.sum(e, axis=1, keepdims=True)
    attn = e / s
    res = jnp.sum(attn * (vg + pos), axis=1)
    out_ref[...] = res + feats_ref[0]


def _bn_body(x_ref, gamma_ref, beta_ref, out_ref):
    x = x_ref[...]
    mean = jnp.mean(x, axis=0, keepdims=True)
    var = jnp.mean((x - mean) ** 2, axis=0, keepdims=True)
    out_ref[...] = (x - mean) / jnp.sqrt(var + 1e-5) * gamma_ref[...] + beta_ref[...]


def _sc_gather(tbl_flat, idx_flat):
    nidx = idx_flat.shape[1]
    window = 128
    mesh = plsc.VectorSubcoreMesh(core_axis_name="c", subcore_axis_name="s")

    @pl.kernel(out_type=jax.ShapeDtypeStruct((nidx, DTBL), tbl_flat.dtype),
               mesh=mesh)
    def kern(tbl_hbm, i_hbm, o_hbm):
        def body(i_vmem, o_vmem):
            pltpu.sync_copy(tbl_hbm.at[i_vmem.at[0]], o_vmem)

        pltpu.emit_pipeline(
            body,
            grid=(nidx // window,),
            in_specs=[pl.BlockSpec((1, window), index_map=lambda i: (0, i))],
            out_specs=[pl.BlockSpec((window, DTBL), index_map=lambda i: (i, 0))],
            core_axis_name=("c", "s"),
            dimension_semantics=(pltpu.PARALLEL,),
        )(i_hbm, o_hbm)

    return kern(tbl_flat, idx_flat)


def kernel(xyz, feats, w_qs, w_ks, w_vs, d1_w, d1_b, d2_w, d2_b,
           g1_w, g1_b, g2_w, g2_b, bn_gamma, bn_beta):
    B, N, _ = xyz.shape
    f32 = jnp.float32
    xyzT = jnp.swapaxes(xyz, 1, 2)
    row = lambda v: v.reshape(1, -1)

    qu, tbl = pl.pallas_call(
        _qkv_body,
        grid=(B,),
        in_specs=[
            pl.BlockSpec((1, N, DM), lambda b: (b, 0, 0)),
            pl.BlockSpec((1, N, 3), lambda b: (b, 0, 0)),
            pl.BlockSpec((DM, DM), lambda b: (0, 0)),
            pl.BlockSpec((DM, DM), lambda b: (0, 0)),
            pl.BlockSpec((DM, DM), lambda b: (0, 0)),
            pl.BlockSpec((3, DM), lambda b: (0, 0)),
            pl.BlockSpec((1, DM), lambda b: (0, 0)),
        ],
        out_specs=[
            pl.BlockSpec((1, N, 2 * DM), lambda b: (b, 0, 0)),
            pl.BlockSpec((1, N, DTBL), lambda b: (b, 0, 0)),
        ],
        out_shape=[
            jax.ShapeDtypeStruct((B, N, 2 * DM), f32),
            jax.ShapeDtypeStruct((B, N, DTBL), jnp.int32),
        ],
    )(feats, xyz, w_qs.T, w_ks.T, w_vs.T, d1_w.T, row(d1_b))

    knn = pl.pallas_call(
        _knn_body,
        grid=(B, N // TILE),
        in_specs=[
            pl.BlockSpec((1, TILE, 3), lambda b, t: (b, t, 0)),
            pl.BlockSpec((1, 3, N), lambda b, t: (b, 0, 0)),
        ],
        out_specs=pl.BlockSpec((1, TILE, KNN), lambda b, t: (b, t, 0)),
        out_shape=jax.ShapeDtypeStruct((B, N, KNN), jnp.int32),
        scratch_shapes=[pltpu.VMEM((TILE, N), f32)],
    )(xyz, xyzT)

    g = _sc_gather(tbl.reshape(B * N, DTBL), knn.reshape(1, B * N * KNN))

    nt = B * N // TILE
    res = pl.pallas_call(
        _attn_body,
        grid=(nt,),
        in_specs=[
            pl.BlockSpec((1, TILE * KNN, DTBL), lambda t: (t, 0, 0)),
            pl.BlockSpec((1, TILE, 2 * DM), lambda t: (t, 0, 0)),
            pl.BlockSpec((1, TILE, DM), lambda t: (t, 0, 0)),
            pl.BlockSpec((DM, DM), lambda t: (0, 0)),
            pl.BlockSpec((1, DM), lambda t: (0, 0)),
            pl.BlockSpec((DM, DM), lambda t: (0, 0)),
            pl.BlockSpec((1, DM), lambda t: (0, 0)),
            pl.BlockSpec((DM, DM), lambda t: (0, 0)),
            pl.BlockSpec((1, DM), lambda t: (0, 0)),
        ],
        out_specs=pl.BlockSpec((TILE, DM), lambda t: (t, 0)),
        out_shape=jax.ShapeDtypeStruct((B * N, DM), f32),
    )(g.reshape(nt, TILE * KNN, DTBL), qu.reshape(nt, TILE, 2 * DM),
      feats.reshape(nt, TILE, DM),
      d2_w.T, row(d2_b),
      g1_w.T, row(g1_b), g2_w.T, row(g2_b))

    out = pl.pallas_call(
        _bn_body,
        grid=(1,),
        in_specs=[
            pl.BlockSpec((B * N, DM), lambda i: (0, 0)),
            pl.BlockSpec((1, DM), lambda i: (0, 0)),
            pl.BlockSpec((1, DM), lambda i: (0, 0)),
        ],
        out_specs=pl.BlockSpec((B * N, DM), lambda i: (0, 0)),
        out_shape=jax.ShapeDtypeStruct((B * N, DM), f32),
    )(res, row(bn_gamma), row(bn_beta))
    return out.reshape(B, N, DM)


# speedup vs baseline: 33.3477x; 1.7760x over previous
import jax
import jax.numpy as jnp
from jax.experimental import pallas as pl
from jax.experimental.pallas import tpu as pltpu
from jax.experimental.pallas import tpu_sc as plsc

KNN = 16
TILE = 512
DM = 64
DTBL = 128


def _bits16(x):
    y = x.astype(jnp.bfloat16).astype(jnp.float32)
    return jax.lax.shift_right_logical(
        jax.lax.bitcast_convert_type(y, jnp.int32), 16)


def _unpack_hi(p):
    return jax.lax.bitcast_convert_type(
        jnp.bitwise_and(p, jnp.int32(-65536)), jnp.float32)


def _unpack_lo(p):
    return jax.lax.bitcast_convert_type(
        jax.lax.shift_left(p, 16), jnp.float32)


def _qkv_body(feats_ref, xyz_ref, wq_ref, wk_ref, wv_ref, d1T_ref, d1b_ref,
              d2b_ref, qu_ref, tbl_ref):
    f = feats_ref[0]
    qu_ref[0, :, 0:DM] = jnp.dot(
        f, wq_ref[...], preferred_element_type=jnp.float32) + d2b_ref[...]
    k = jnp.dot(f, wk_ref[...], preferred_element_type=jnp.float32)
    v = jnp.dot(f, wv_ref[...], preferred_element_type=jnp.float32) + d2b_ref[...]
    x = xyz_ref[0]
    r0 = d1T_ref[0:1, :]
    r1 = d1T_ref[1:2, :]
    r2 = d1T_ref[2:3, :]
    w = x[:, 0:1] * r0 + x[:, 1:2] * r1 + x[:, 2:3] * r2
    qu_ref[0, :, DM:2 * DM] = w + d1b_ref[...]
    whi = w.astype(jnp.bfloat16).astype(jnp.float32)
    wlo = w - whi
    tbl_ref[0, :, 0:DM] = jnp.bitwise_or(
        jax.lax.shift_left(_bits16(k), 16), _bits16(v))
    tbl_ref[0, :, DM:DTBL] = jnp.bitwise_or(
        jax.lax.shift_left(_bits16(whi), 16), _bits16(wlo))


_BATCHER16 = [
    (0, 1), (2, 3), (4, 5), (6, 7), (8, 9), (10, 11), (12, 13), (14, 15),
    (0, 2), (1, 3), (4, 6), (5, 7), (8, 10), (9, 11), (12, 14), (13, 15),
    (1, 2), (5, 6), (9, 10), (13, 14), (0, 4), (1, 5), (2, 6), (3, 7),
    (8, 12), (9, 13), (10, 14), (11, 15), (2, 4), (3, 5), (10, 12), (11, 13),
    (1, 2), (3, 4), (5, 6), (9, 10), (11, 12), (13, 14), (0, 8), (1, 9),
    (2, 10), (3, 11), (4, 12), (5, 13), (6, 14), (7, 15), (4, 8), (5, 9),
    (6, 10), (7, 11), (2, 4), (3, 5), (6, 8), (7, 9), (10, 12), (11, 13),
    (1, 2), (3, 4), (5, 6), (7, 8), (9, 10), (11, 12), (13, 14),
]

_NCHUNK = 16
_CW = 128


def _knn_body(xyzq_ref, xyzT_ref, knn_ref, lvl_ref):
    n = xyzT_ref.shape[2]
    b = pl.program_id(0)
    i32 = jnp.int32
    f32 = jnp.float32
    xq = xyzq_ref[0]
    xk = xyzT_ref[0]
    dx = xq[:, 0:1] - xk[0:1, :]
    dy = xq[:, 1:2] - xk[1:2, :]
    dz = xq[:, 2:3] - xk[2:3, :]
    d = dx * dx + dy * dy + dz * dz
    bits = jnp.bitwise_and(
        jax.lax.bitcast_convert_type(d + 1.0, i32), i32(-16))
    for s in range(_NCHUNK):
        lvl_ref[s] = jax.lax.bitcast_convert_type(
            jnp.bitwise_or(bits[:, s * _CW:(s + 1) * _CW], i32(s)), f32)
    for (a, c) in _BATCHER16:
        x = lvl_ref[a]
        y = lvl_ref[c]
        lvl_ref[a] = jnp.minimum(x, y)
        lvl_ref[c] = jnp.maximum(x, y)
    lane = jax.lax.broadcasted_iota(i32, (TILE, _CW), 1)
    lanef = lane.astype(f32)
    base = b * n
    for t in range(KNN):
        h = lvl_ref[0]
        m = jnp.min(h, axis=1, keepdims=True)
        cf = jnp.min(jnp.where(h == m, lanef, f32(_CW)), axis=1, keepdims=True)
        c = cf.astype(i32)
        s = jnp.bitwise_and(jax.lax.bitcast_convert_type(m, i32), i32(15))
        knn_ref[0, :, t:t + 1] = base + s * _CW + c
        if t < KNN - 1:
            onehot = lane == c
            for l in range(KNN - 1 - t):
                lvl_ref[l] = jnp.where(onehot, lvl_ref[l + 1], lvl_ref[l])


def _attn_body(g_ref, qu_ref, feats_ref,
               d2T_ref,
               g1T_ref, g1b_ref, g2T_ref, out_ref):
    G = g_ref[0]
    p1 = G[:, 0:DM]
    p2 = G[:, DM:DTBL]
    kg = _unpack_hi(p1).reshape(TILE, KNN, DM)
    vg = _unpack_lo(p1).reshape(TILE, KNN, DM)
    wg = (_unpack_hi(p2) + _unpack_lo(p2)).reshape(TILE, KNN, DM)
    qu = qu_ref[0]
    q3 = qu[:, 0:DM][:, None, :]
    u3 = qu[:, DM:2 * DM][:, None, :]
    pre = u3 - wg
    pe = jax.nn.relu(pre).reshape(TILE * KNN, DM)
    pos = jnp.dot(pe, d2T_ref[...],
                  preferred_element_type=jnp.float32).reshape(TILE, KNN, DM)
    h = q3 - kg + pos
    a1 = jax.nn.relu(jnp.dot(h.reshape(TILE * KNN, DM), g1T_ref[...],
                             preferred_element_type=jnp.float32) + g1b_ref[...])
    logits = jnp.dot(a1, g2T_ref[...],
                     preferred_element_type=jnp.float32).reshape(TILE, KNN, DM)
    e = jnp.exp(logits)
    s = jnp.sum(e, axis=1)
    acc = jnp.sum(e * (vg + pos), axis=1)
    out_ref[...] = acc / s + feats_ref[0]


def _bn_body(x_ref, gamma_ref, beta_ref, out_ref):
    x = x_ref[...]
    mean = jnp.mean(x, axis=0, keepdims=True)
    var = jnp.mean((x - mean) ** 2, axis=0, keepdims=True)
    out_ref[...] = (x - mean) / jnp.sqrt(var + 1e-5) * gamma_ref[...] + beta_ref[...]


def _sc_gather(tbl_flat, idx_flat):
    nidx = idx_flat.shape[1]
    window = 128
    mesh = plsc.VectorSubcoreMesh(core_axis_name="c", subcore_axis_name="s")

    @pl.kernel(out_type=jax.ShapeDtypeStruct((nidx, DTBL), tbl_flat.dtype),
               mesh=mesh)
    def kern(tbl_hbm, i_hbm, o_hbm):
        def body(i_vmem, o_vmem):
            pltpu.sync_copy(tbl_hbm.at[i_vmem.at[0]], o_vmem)

        pltpu.emit_pipeline(
            body,
            grid=(nidx // window,),
            in_specs=[pl.BlockSpec((1, window), index_map=lambda i: (0, i))],
            out_specs=[pl.BlockSpec((window, DTBL), index_map=lambda i: (i, 0))],
            core_axis_name=("c", "s"),
            dimension_semantics=(pltpu.PARALLEL,),
        )(i_hbm, o_hbm)

    return kern(tbl_flat, idx_flat)


def kernel(xyz, feats, w_qs, w_ks, w_vs, d1_w, d1_b, d2_w, d2_b,
           g1_w, g1_b, g2_w, g2_b, bn_gamma, bn_beta):
    B, N, _ = xyz.shape
    f32 = jnp.float32
    xyzT = jnp.swapaxes(xyz, 1, 2)
    row = lambda v: v.reshape(1, -1)

    qu, tbl = pl.pallas_call(
        _qkv_body,
        grid=(B,),
        in_specs=[
            pl.BlockSpec((1, N, DM), lambda b: (b, 0, 0)),
            pl.BlockSpec((1, N, 3), lambda b: (b, 0, 0)),
            pl.BlockSpec((DM, DM), lambda b: (0, 0)),
            pl.BlockSpec((DM, DM), lambda b: (0, 0)),
            pl.BlockSpec((DM, DM), lambda b: (0, 0)),
            pl.BlockSpec((3, DM), lambda b: (0, 0)),
            pl.BlockSpec((1, DM), lambda b: (0, 0)),
            pl.BlockSpec((1, DM), lambda b: (0, 0)),
        ],
        out_specs=[
            pl.BlockSpec((1, N, 2 * DM), lambda b: (b, 0, 0)),
            pl.BlockSpec((1, N, DTBL), lambda b: (b, 0, 0)),
        ],
        out_shape=[
            jax.ShapeDtypeStruct((B, N, 2 * DM), f32),
            jax.ShapeDtypeStruct((B, N, DTBL), jnp.int32),
        ],
    )(feats, xyz, w_qs.T, w_ks.T, w_vs.T, d1_w.T, row(d1_b), row(d2_b))

    nb = N // TILE
    knns = [
        pl.pallas_call(
            _knn_body,
            grid=(1, nb),
            in_specs=[
                pl.BlockSpec((1, TILE, 3), lambda b, t: (b, t, 0)),
                pl.BlockSpec((1, 3, N), lambda b, t: (b, 0, 0)),
            ],
            out_specs=pl.BlockSpec((1, TILE, KNN), lambda b, t: (b, t, 0)),
            out_shape=jax.ShapeDtypeStruct((1, N, KNN), jnp.int32),
            scratch_shapes=[pltpu.VMEM((_NCHUNK, TILE, _CW), f32)],
        )(jax.lax.slice_in_dim(xyz, b, b + 1, axis=0),
          jax.lax.slice_in_dim(xyzT, b, b + 1, axis=0))
        for b in range(B)
    ]
    gs = [
        _sc_gather(tbl[b], knns[b].reshape(1, N * KNN))
        for b in range(B)
    ]

    def attn_call(g_b, qu_b, feats_b):
        nt = N // TILE
        return pl.pallas_call(
            _attn_body,
            grid=(nt,),
            in_specs=[
                pl.BlockSpec((1, TILE * KNN, DTBL), lambda t: (t, 0, 0)),
                pl.BlockSpec((1, TILE, 2 * DM), lambda t: (t, 0, 0)),
                pl.BlockSpec((1, TILE, DM), lambda t: (t, 0, 0)),
                pl.BlockSpec((DM, DM), lambda t: (0, 0)),
                pl.BlockSpec((DM, DM), lambda t: (0, 0)),
                pl.BlockSpec((1, DM), lambda t: (0, 0)),
                pl.BlockSpec((DM, DM), lambda t: (0, 0)),
            ],
            out_specs=pl.BlockSpec((TILE, DM), lambda t: (t, 0)),
            out_shape=jax.ShapeDtypeStruct((N, DM), f32),
        )(g_b.reshape(nt, TILE * KNN, DTBL), qu_b.reshape(nt, TILE, 2 * DM),
          feats_b.reshape(nt, TILE, DM),
          d2_w.T, g1_w.T, row(g1_b), g2_w.T)

    res = jnp.concatenate(
        [attn_call(gs[b], qu[b], feats[b]) for b in range(B)], axis=0)

    out = pl.pallas_call(
        _bn_body,
        grid=(1,),
        in_specs=[
            pl.BlockSpec((B * N, DM), lambda i: (0, 0)),
            pl.BlockSpec((1, DM), lambda i: (0, 0)),
            pl.BlockSpec((1, DM), lambda i: (0, 0)),
        ],
        out_specs=pl.BlockSpec((B * N, DM), lambda i: (0, 0)),
        out_shape=jax.ShapeDtypeStruct((B * N, DM), f32),
    )(res, row(bn_gamma), row(bn_beta))
    return out.reshape(B, N, DM)


# speedup vs baseline: 33.4091x; 1.0018x over previous
import jax
import jax.numpy as jnp
from jax.experimental import pallas as pl
from jax.experimental.pallas import tpu as pltpu
from jax.experimental.pallas import tpu_sc as plsc

KNN = 16
TILE = 512
DM = 64
DTBL = 128


def _bits16(x):
    y = x.astype(jnp.bfloat16).astype(jnp.float32)
    return jax.lax.shift_right_logical(
        jax.lax.bitcast_convert_type(y, jnp.int32), 16)


def _unpack_hi(p):
    return jax.lax.bitcast_convert_type(
        jnp.bitwise_and(p, jnp.int32(-65536)), jnp.float32)


def _unpack_lo(p):
    return jax.lax.bitcast_convert_type(
        jax.lax.shift_left(p, 16), jnp.float32)


def _qkv_body(feats_ref, xyz_ref, wq_ref, wk_ref, wv_ref, d1T_ref, d1b_ref,
              d2b_ref, qu_ref, tbl_ref):
    f = feats_ref[0]
    qu_ref[0, :, 0:DM] = jnp.dot(
        f, wq_ref[...], preferred_element_type=jnp.float32) + d2b_ref[...]
    k = jnp.dot(f, wk_ref[...], preferred_element_type=jnp.float32)
    v = jnp.dot(f, wv_ref[...], preferred_element_type=jnp.float32) + d2b_ref[...]
    x = xyz_ref[0]
    r0 = d1T_ref[0:1, :]
    r1 = d1T_ref[1:2, :]
    r2 = d1T_ref[2:3, :]
    w = x[:, 0:1] * r0 + x[:, 1:2] * r1 + x[:, 2:3] * r2
    qu_ref[0, :, DM:2 * DM] = w + d1b_ref[...]
    whi = w.astype(jnp.bfloat16).astype(jnp.float32)
    wlo = w - whi
    tbl_ref[0, :, 0:DM] = jnp.bitwise_or(
        jax.lax.shift_left(_bits16(k), 16), _bits16(v))
    tbl_ref[0, :, DM:DTBL] = jnp.bitwise_or(
        jax.lax.shift_left(_bits16(whi), 16), _bits16(wlo))


_BATCHER16 = [
    (0, 1), (2, 3), (4, 5), (6, 7), (8, 9), (10, 11), (12, 13), (14, 15),
    (0, 2), (1, 3), (4, 6), (5, 7), (8, 10), (9, 11), (12, 14), (13, 15),
    (1, 2), (5, 6), (9, 10), (13, 14), (0, 4), (1, 5), (2, 6), (3, 7),
    (8, 12), (9, 13), (10, 14), (11, 15), (2, 4), (3, 5), (10, 12), (11, 13),
    (1, 2), (3, 4), (5, 6), (9, 10), (11, 12), (13, 14), (0, 8), (1, 9),
    (2, 10), (3, 11), (4, 12), (5, 13), (6, 14), (7, 15), (4, 8), (5, 9),
    (6, 10), (7, 11), (2, 4), (3, 5), (6, 8), (7, 9), (10, 12), (11, 13),
    (1, 2), (3, 4), (5, 6), (7, 8), (9, 10), (11, 12), (13, 14),
]

_NCHUNK = 16
_CW = 128


def _knn_body(xyzq_ref, xyzT_ref, knn_ref, lvl_ref):
    n = xyzT_ref.shape[2]
    b = pl.program_id(0)
    i32 = jnp.int32
    f32 = jnp.float32
    xq = xyzq_ref[0]
    xk = xyzT_ref[0]
    dx = xq[:, 0:1] - xk[0:1, :]
    dy = xq[:, 1:2] - xk[1:2, :]
    dz = xq[:, 2:3] - xk[2:3, :]
    d = dx * dx + dy * dy + dz * dz
    bits = jnp.bitwise_and(
        jax.lax.bitcast_convert_type(d + 1.0, i32), i32(-16))
    for s in range(_NCHUNK):
        lvl_ref[s] = jax.lax.bitcast_convert_type(
            jnp.bitwise_or(bits[:, s * _CW:(s + 1) * _CW], i32(s)), f32)
    for (a, c) in _BATCHER16:
        x = lvl_ref[a]
        y = lvl_ref[c]
        lvl_ref[a] = jnp.minimum(x, y)
        lvl_ref[c] = jnp.maximum(x, y)
    lane = jax.lax.broadcasted_iota(i32, (TILE, _CW), 1)
    lanef = lane.astype(f32)
    base = b * n
    for t in range(KNN):
        h = lvl_ref[0]
        m = jnp.min(h, axis=1, keepdims=True)
        cf = jnp.min(jnp.where(h == m, lanef, f32(_CW)), axis=1, keepdims=True)
        c = cf.astype(i32)
        s = jnp.bitwise_and(jax.lax.bitcast_convert_type(m, i32), i32(15))
        knn_ref[0, :, t:t + 1] = base + s * _CW + c
        if t < KNN - 1:
            onehot = lane == c
            for l in range(KNN - 1 - t):
                lvl_ref[l] = jnp.where(onehot, lvl_ref[l + 1], lvl_ref[l])


def _attn_body(g_ref, qu_ref, feats_ref,
               d2T_ref,
               g1T_ref, g1b_ref, g2T_ref, out_ref):
    G = g_ref[0]
    p1 = G[:, 0:DM]
    p2 = G[:, DM:DTBL]
    kg = _unpack_hi(p1).reshape(TILE, KNN, DM)
    vg = _unpack_lo(p1).reshape(TILE, KNN, DM)
    wg = (_unpack_hi(p2) + _unpack_lo(p2)).reshape(TILE, KNN, DM)
    qu = qu_ref[0]
    q3 = qu[:, 0:DM][:, None, :]
    u3 = qu[:, DM:2 * DM][:, None, :]
    pre = u3 - wg
    pe = jax.nn.relu(pre).reshape(TILE * KNN, DM)
    pos = jnp.dot(pe, d2T_ref[...],
                  preferred_element_type=jnp.float32).reshape(TILE, KNN, DM)
    h = q3 - kg + pos
    a1 = jax.nn.relu(jnp.dot(h.reshape(TILE * KNN, DM), g1T_ref[...],
                             preferred_element_type=jnp.float32) + g1b_ref[...])
    logits = jnp.dot(a1, g2T_ref[...],
                     preferred_element_type=jnp.float32).reshape(TILE, KNN, DM)
    e = jnp.exp(logits)
    s = jnp.sum(e, axis=1)
    acc = jnp.sum(e * (vg + pos), axis=1)
    out_ref[...] = acc / s + feats_ref[0]


def _bn_body(x_ref, gamma_ref, beta_ref, out_ref):
    x = x_ref[...]
    mean = jnp.mean(x, axis=0, keepdims=True)
    var = jnp.mean((x - mean) ** 2, axis=0, keepdims=True)
    out_ref[...] = (x - mean) / jnp.sqrt(var + 1e-5) * gamma_ref[...] + beta_ref[...]


def _sc_gather(tbl_flat, idx_flat):
    nidx = idx_flat.shape[1]
    window = 256
    mesh = plsc.VectorSubcoreMesh(core_axis_name="c", subcore_axis_name="s")

    @pl.kernel(out_type=jax.ShapeDtypeStruct((nidx, DTBL), tbl_flat.dtype),
               mesh=mesh)
    def kern(tbl_hbm, i_hbm, o_hbm):
        def body(i_vmem, o_vmem):
            pltpu.sync_copy(tbl_hbm.at[i_vmem.at[0]], o_vmem)

        pltpu.emit_pipeline(
            body,
            grid=(nidx // window,),
            in_specs=[pl.BlockSpec((1, window), index_map=lambda i: (0, i))],
            out_specs=[pl.BlockSpec((window, DTBL), index_map=lambda i: (i, 0))],
            core_axis_name=("c", "s"),
            dimension_semantics=(pltpu.PARALLEL,),
        )(i_hbm, o_hbm)

    return kern(tbl_flat, idx_flat)


def kernel(xyz, feats, w_qs, w_ks, w_vs, d1_w, d1_b, d2_w, d2_b,
           g1_w, g1_b, g2_w, g2_b, bn_gamma, bn_beta):
    B, N, _ = xyz.shape
    f32 = jnp.float32
    xyzT = jnp.swapaxes(xyz, 1, 2)
    row = lambda v: v.reshape(1, -1)

    qu, tbl = pl.pallas_call(
        _qkv_body,
        grid=(B,),
        in_specs=[
            pl.BlockSpec((1, N, DM), lambda b: (b, 0, 0)),
            pl.BlockSpec((1, N, 3), lambda b: (b, 0, 0)),
            pl.BlockSpec((DM, DM), lambda b: (0, 0)),
            pl.BlockSpec((DM, DM), lambda b: (0, 0)),
            pl.BlockSpec((DM, DM), lambda b: (0, 0)),
            pl.BlockSpec((3, DM), lambda b: (0, 0)),
            pl.BlockSpec((1, DM), lambda b: (0, 0)),
            pl.BlockSpec((1, DM), lambda b: (0, 0)),
        ],
        out_specs=[
            pl.BlockSpec((1, N, 2 * DM), lambda b: (b, 0, 0)),
            pl.BlockSpec((1, N, DTBL), lambda b: (b, 0, 0)),
        ],
        out_shape=[
            jax.ShapeDtypeStruct((B, N, 2 * DM), f32),
            jax.ShapeDtypeStruct((B, N, DTBL), jnp.int32),
        ],
    )(feats, xyz, w_qs.T, w_ks.T, w_vs.T, d1_w.T, row(d1_b), row(d2_b))

    nb = N // TILE
    knns = [
        pl.pallas_call(
            _knn_body,
            grid=(1, nb),
            in_specs=[
                pl.BlockSpec((1, TILE, 3), lambda b, t: (b, t, 0)),
                pl.BlockSpec((1, 3, N), lambda b, t: (b, 0, 0)),
            ],
            out_specs=pl.BlockSpec((1, TILE, KNN), lambda b, t: (b, t, 0)),
            out_shape=jax.ShapeDtypeStruct((1, N, KNN), jnp.int32),
            scratch_shapes=[pltpu.VMEM((_NCHUNK, TILE, _CW), f32)],
        )(jax.lax.slice_in_dim(xyz, b, b + 1, axis=0),
          jax.lax.slice_in_dim(xyzT, b, b + 1, axis=0))
        for b in range(B)
    ]
    gs = [
        _sc_gather(tbl[b], knns[b].reshape(1, N * KNN))
        for b in range(B)
    ]

    def attn_call(g_b, qu_b, feats_b):
        nt = N // TILE
        return pl.pallas_call(
            _attn_body,
            grid=(nt,),
            in_specs=[
                pl.BlockSpec((1, TILE * KNN, DTBL), lambda t: (t, 0, 0)),
                pl.BlockSpec((1, TILE, 2 * DM), lambda t: (t, 0, 0)),
                pl.BlockSpec((1, TILE, DM), lambda t: (t, 0, 0)),
                pl.BlockSpec((DM, DM), lambda t: (0, 0)),
                pl.BlockSpec((DM, DM), lambda t: (0, 0)),
                pl.BlockSpec((1, DM), lambda t: (0, 0)),
                pl.BlockSpec((DM, DM), lambda t: (0, 0)),
            ],
            out_specs=pl.BlockSpec((TILE, DM), lambda t: (t, 0)),
            out_shape=jax.ShapeDtypeStruct((N, DM), f32),
        )(g_b.reshape(nt, TILE * KNN, DTBL), qu_b.reshape(nt, TILE, 2 * DM),
          feats_b.reshape(nt, TILE, DM),
          d2_w.T, g1_w.T, row(g1_b), g2_w.T)

    res = jnp.concatenate(
        [attn_call(gs[b], qu[b], feats[b]) for b in range(B)], axis=0)

    out = pl.pallas_call(
        _bn_body,
        grid=(1,),
        in_specs=[
            pl.BlockSpec((B * N, DM), lambda i: (0, 0)),
            pl.BlockSpec((1, DM), lambda i: (0, 0)),
            pl.BlockSpec((1, DM), lambda i: (0, 0)),
        ],
        out_specs=pl.BlockSpec((B * N, DM), lambda i: (0, 0)),
        out_shape=jax.ShapeDtypeStruct((B * N, DM), f32),
    )(res, row(bn_gamma), row(bn_beta))
    return out.reshape(B, N, DM)
